# baseline (device time: 65910 ns/iter reference)
import jax
import jax.numpy as jnp
from jax import lax
from jax.experimental import pallas as pl
from jax.experimental.pallas import tpu as pltpu

N_DEV = 4
N_TOK = 2048
D = 512
H = 1024
E_LOCAL = 4
E_TOTAL = 16
CHUNK = N_TOK // N_DEV


def kernel(x, router_W, route_idx, expert_W, shared_W):
    def body(x_ref, rw_ref, idx_ref, ew_ref, sw_ref, out_ref,
             acc_ref, recv_ref, send_sems, recv_sems):
        my = lax.axis_index("i")
        left = lax.rem(my - 1 + N_DEV, N_DEV)
        right = lax.rem(my + 1, N_DEV)

        barrier_sem = pltpu.get_barrier_semaphore()
        for nbr in (left, right):
            pl.semaphore_signal(
                barrier_sem, inc=1,
                device_id=(nbr,), device_id_type=pl.DeviceIdType.MESH,
            )
        pl.semaphore_wait(barrier_sem, 2)

        xf = x_ref[...]
        scores = jnp.dot(xf, rw_ref[...], preferred_element_type=jnp.float32)
        s_max = jnp.max(scores, axis=-1, keepdims=True)
        p = jnp.exp(scores - s_max)
        probs = p / jnp.sum(p, axis=-1, keepdims=True)
        idx = idx_ref[...]
        e_iota = lax.broadcasted_iota(jnp.int32, (N_TOK, E_TOTAL), 1)
        sel = jnp.sum(jnp.where(e_iota == idx, probs, 0.0), axis=-1,
                      keepdims=True)

        partial = jnp.zeros((N_TOK, H), jnp.float32)
        for j in range(E_LOCAL):
            e_glob = my * E_LOCAL + j
            gate = jnp.where(idx == e_glob, sel, 0.0)
            xg = (xf * gate).astype(jnp.bfloat16)
            w = ew_ref[j].astype(jnp.bfloat16)
            partial = partial + jnp.dot(
                xg, w, preferred_element_type=jnp.float32)
        acc_ref[...] = partial.reshape(N_DEV, CHUNK, H).astype(jnp.bfloat16)

        for s in range(N_DEV - 1):
            send_c = lax.rem(my - s - 1 + N_DEV, N_DEV)
            recv_c = lax.rem(my - s - 2 + 2 * N_DEV, N_DEV)
            rdma = pltpu.make_async_remote_copy(
                src_ref=acc_ref.at[send_c],
                dst_ref=recv_ref.at[s],
                send_sem=send_sems.at[s],
                recv_sem=recv_sems.at[s],
                device_id=(right,),
                device_id_type=pl.DeviceIdType.MESH,
            )
            rdma.start()
            rdma.wait()
            acc_ref[recv_c] = acc_ref[recv_c] + recv_ref[s]

        xs = x_ref[pl.ds(my * CHUNK, CHUNK), :].astype(jnp.bfloat16)
        shared = jnp.dot(xs, sw_ref[...].astype(jnp.bfloat16),
                         preferred_element_type=jnp.float32)
        out_ref[...] = shared + acc_ref[my].astype(jnp.float32)

    return pl.pallas_call(
        body,
        out_shape=jax.ShapeDtypeStruct((CHUNK, H), jnp.float32),
        in_specs=[pl.BlockSpec(memory_space=pltpu.VMEM)] * 5,
        out_specs=pl.BlockSpec(memory_space=pltpu.VMEM),
        scratch_shapes=[
            pltpu.VMEM((N_DEV, CHUNK, H), jnp.bfloat16),
            pltpu.VMEM((N_DEV - 1, CHUNK, H), jnp.bfloat16),
            pltpu.SemaphoreType.DMA((N_DEV - 1,)),
            pltpu.SemaphoreType.DMA((N_DEV - 1,)),
        ],
        compiler_params=pltpu.CompilerParams(collective_id=0),
    )(x, router_W, route_idx, expert_W, shared_W)


# device time: 44500 ns/iter; 1.4811x vs baseline; 1.4811x over previous
import jax
import jax.numpy as jnp
from jax import lax
from jax.experimental import pallas as pl
from jax.experimental.pallas import tpu as pltpu

N_DEV = 4
N_TOK = 2048
D = 512
H = 1024
E_LOCAL = 4
E_TOTAL = 16
CHUNK = N_TOK // N_DEV


def kernel(x, router_W, route_idx, expert_W, shared_W):
    def body(x_ref, rw_ref, idx_ref, ew_ref, sw_ref, out_ref,
             send_ref, recv_ref, send_sems, recv_sems):
        my = lax.axis_index("i")
        right = lax.rem(my + 1, N_DEV)
        left = lax.rem(my - 1 + N_DEV, N_DEV)
        opp = lax.rem(my + 2, N_DEV)

        barrier_sem = pltpu.get_barrier_semaphore()
        for nbr in (left, right, opp):
            pl.semaphore_signal(
                barrier_sem, inc=1,
                device_id=(nbr,), device_id_type=pl.DeviceIdType.MESH,
            )
        pl.semaphore_wait(barrier_sem, N_DEV - 1)

        e_iota = lax.broadcasted_iota(jnp.int32, (CHUNK, E_TOTAL), 1)

        def chunk_partial(c):
            xc = x_ref[pl.ds(c * CHUNK, CHUNK), :]
            idc = idx_ref[pl.ds(c * CHUNK, CHUNK), :]
            scores = jnp.dot(xc, rw_ref[...],
                             preferred_element_type=jnp.float32)
            s_max = jnp.max(scores, axis=-1, keepdims=True)
            p = jnp.exp(scores - s_max)
            probs = p / jnp.sum(p, axis=-1, keepdims=True)
            sel = jnp.sum(jnp.where(e_iota == idc, probs, 0.0), axis=-1,
                          keepdims=True)
            xg_all = xc * sel
            acc = jnp.zeros((CHUNK, H), jnp.float32)
            for j in range(E_LOCAL):
                e_glob = my * E_LOCAL + j
                xg = jnp.where(idc == e_glob, xg_all, 0.0).astype(jnp.bfloat16)
                w = ew_ref[j].astype(jnp.bfloat16)
                acc = acc + jnp.dot(xg, w,
                                    preferred_element_type=jnp.float32)
            return acc

        rdmas = []
        for slot, dest in enumerate((right, left, opp)):
            send_ref[slot] = chunk_partial(dest).astype(jnp.bfloat16)
            rdma = pltpu.make_async_remote_copy(
                src_ref=send_ref.at[slot],
                dst_ref=recv_ref.at[slot],
                send_sem=send_sems.at[slot],
                recv_sem=recv_sems.at[slot],
                device_id=(dest,),
                device_id_type=pl.DeviceIdType.MESH,
            )
            rdma.start()
            rdmas.append(rdma)

        total = chunk_partial(my)
        xs = x_ref[pl.ds(my * CHUNK, CHUNK), :].astype(jnp.bfloat16)
        total = total + jnp.dot(xs, sw_ref[...].astype(jnp.bfloat16),
                                preferred_element_type=jnp.float32)

        for slot in range(N_DEV - 1):
            rdmas[slot].wait_recv()
            total = total + recv_ref[slot].astype(jnp.float32)
        out_ref[...] = total
        for slot in range(N_DEV - 1):
            rdmas[slot].wait_send()

    return pl.pallas_call(
        body,
        out_shape=jax.ShapeDtypeStruct((CHUNK, H), jnp.float32),
        in_specs=[pl.BlockSpec(memory_space=pltpu.VMEM)] * 5,
        out_specs=pl.BlockSpec(memory_space=pltpu.VMEM),
        scratch_shapes=[
            pltpu.VMEM((N_DEV - 1, CHUNK, H), jnp.bfloat16),
            pltpu.VMEM((N_DEV - 1, CHUNK, H), jnp.bfloat16),
            pltpu.SemaphoreType.DMA((N_DEV - 1,)),
            pltpu.SemaphoreType.DMA((N_DEV - 1,)),
        ],
        compiler_params=pltpu.CompilerParams(collective_id=0),
    )(x, router_W, route_idx, expert_W, shared_W)
